# baseline (speedup 1.0000x reference)
import functools

import jax
import jax.numpy as jnp
from jax import lax
from jax.experimental import pallas as pl
from jax.experimental.pallas import tpu as pltpu
from jax.experimental.pallas import tpu_sc as plsc

_NBUF = 4


def _make_gather(S, T, V, D, NC, NS):
    NW = NC * NS
    s_per_w = S // NW
    nbuf = _NBUF
    n_outer = s_per_w // nbuf
    assert S % NW == 0 and s_per_w % nbuf == 0
    mesh = plsc.VectorSubcoreMesh(core_axis_name="c", subcore_axis_name="s")

    @functools.partial(
        pl.kernel,
        mesh=mesh,
        out_type=jax.ShapeDtypeStruct((S, T, D), jnp.float32),
        compiler_params=pltpu.CompilerParams(use_tc_tiling_on_sc=False),
        scratch_types=[
            pltpu.VMEM((s_per_w, T), jnp.int32),
            pltpu.VMEM((nbuf, T, D), jnp.float32),
            [pltpu.SemaphoreType.DMA] * _NBUF,
            [pltpu.SemaphoreType.DMA] * _NBUF,
        ],
    )
    def gather_kernel(table_hbm, idx_hbm, out_hbm, idx_v, rows_v, gsems, ssems):
        wid = lax.axis_index("s") * NC + lax.axis_index("c")
        base = wid * s_per_w
        pltpu.sync_copy(idx_hbm.at[pl.ds(base, s_per_w)], idx_v)

        def start_gather(j, b):
            pltpu.async_copy(table_hbm.at[idx_v.at[j]], rows_v.at[b], gsems[b])

        def wait_gather(b):
            pltpu.make_async_copy(
                table_hbm.at[idx_v.at[0]], rows_v.at[b], gsems[b]
            ).wait()

        def start_store(j, b):
            pltpu.async_copy(rows_v.at[b], out_hbm.at[base + j], ssems[b])

        def wait_store(b):
            pltpu.make_async_copy(rows_v.at[b], out_hbm.at[base], ssems[b]).wait()

        for b in range(nbuf):
            start_gather(b, b)

        def outer(go, carry):
            for b in range(nbuf):
                j = go * nbuf + b
                wait_gather(b)
                start_store(j, b)
                wait_store(b)
                start_gather(j + nbuf, b)
            return carry

        lax.fori_loop(0, n_outer - 1, outer, 0)

        for b in range(nbuf):
            j = (n_outer - 1) * nbuf + b
            wait_gather(b)
            start_store(j, b)
        for b in range(nbuf):
            wait_store(b)

    return gather_kernel


def kernel(token_ids, W):
    S, T = token_ids.shape
    V, D = W.shape
    info = plsc.get_sparse_core_info()
    NC, NS = info.num_cores, info.num_subcores
    return _make_gather(S, T, V, D, NC, NS)(W, token_ids)


# speedup vs baseline: 1.2234x; 1.2234x over previous
import functools

import jax
import jax.numpy as jnp
from jax import lax
from jax.experimental import pallas as pl
from jax.experimental.pallas import tpu as pltpu
from jax.experimental.pallas import tpu_sc as plsc

_NBUF = 4


def _make_gather(S, T, V, D, NC, NS):
    NW = NC * NS
    s_per_w = S // NW
    nbuf = _NBUF
    n_outer = s_per_w // nbuf
    mesh = plsc.VectorSubcoreMesh(core_axis_name="c", subcore_axis_name="s")

    @functools.partial(
        pl.kernel,
        mesh=mesh,
        out_type=jax.ShapeDtypeStruct((S, T, 2 * D), jnp.float32),
        scratch_types=[
            pltpu.VMEM((s_per_w * T,), jnp.int32),
            pltpu.VMEM((nbuf * T, 2 * D), jnp.float32),
            [pltpu.SemaphoreType.DMA] * _NBUF,
            [pltpu.SemaphoreType.DMA] * _NBUF,
        ],
    )
    def gather_kernel(table_hbm, idx_hbm, out_hbm, idx_v, rows_v, gsems, ssems):
        wid = lax.axis_index("s") * NC + lax.axis_index("c")
        base = wid * s_per_w
        pltpu.sync_copy(idx_hbm.at[pl.ds(base * T, s_per_w * T)], idx_v)

        def start_gather(j, b):
            pltpu.async_copy(
                table_hbm.at[idx_v.at[pl.ds(j * T, T)]],
                rows_v.at[pl.ds(b * T, T)],
                gsems[b],
            )

        def wait_gather(b):
            pltpu.make_async_copy(
                table_hbm.at[idx_v.at[pl.ds(0, T)]],
                rows_v.at[pl.ds(0, T)],
                gsems[b],
            ).wait()

        def start_store(j, b):
            pltpu.async_copy(
                rows_v.at[pl.ds(b * T, T)],
                out_hbm.at[base + j],
                ssems[b],
            )

        def wait_store(b):
            pltpu.make_async_copy(
                rows_v.at[pl.ds(0, T)], out_hbm.at[base], ssems[b]
            ).wait()

        for b in range(nbuf):
            start_gather(b, b)

        def outer(go, carry):
            for b in range(nbuf):
                j = go * nbuf + b
                wait_gather(b)
                start_store(j, b)
                wait_store(b)
                start_gather(j + nbuf, b)
            return carry

        lax.fori_loop(0, n_outer - 1, outer, 0)

        for b in range(nbuf):
            j = (n_outer - 1) * nbuf + b
            wait_gather(b)
            start_store(j, b)
        for b in range(nbuf):
            wait_store(b)

    return gather_kernel


def kernel(token_ids, W):
    S, T = token_ids.shape
    V, D = W.shape
    info = plsc.get_sparse_core_info()
    NC, NS = info.num_cores, info.num_subcores
    Wp = jnp.pad(W, ((0, 0), (0, D)))
    idx_flat = token_ids.reshape(S * T)
    out_wide = _make_gather(S, T, V, D, NC, NS)(Wp, idx_flat)
    return out_wide[:, :, :D]
